# baseline (speedup 1.0000x reference)
import functools

import jax
import jax.numpy as jnp
from jax import lax
from jax.experimental import pallas as pl
from jax.experimental.pallas import tpu as pltpu
from jax.experimental.pallas import tpu_sc as plsc

_K = 8192
_D = 256
_T = 256
_N = 4096
_NT = _N // _T


def _tc_body(z_ref, emb_ref, idx_ref, loss_ref, emb_m2, esq_s, loss_acc):
    i = pl.program_id(0)

    @pl.when(i == 0)
    def _():
        emb = emb_ref[...]
        emb_m2[...] = emb * -2.0
        esq_s[...] = jnp.sum(emb * emb, axis=1, keepdims=True)
        loss_acc[0, 0] = 0.0

    zb = z_ref[0]
    zsq = jnp.sum(zb * zb, axis=0, keepdims=True)
    mm2 = lax.dot_general(emb_m2[...], zb, (((1,), (0,)), ((), ())),
                          preferred_element_type=jnp.float32)
    d = (zsq + esq_s[...]) + mm2

    bmin = jnp.min(d, axis=0, keepdims=True)
    idx_ref[0] = jnp.argmin(d, axis=0).astype(jnp.int32).reshape(1, _T)
    loss_acc[0, 0] += jnp.sum(bmin)

    @pl.when(i == _NT - 1)
    def _():
        loss_ref[0, 0] = loss_acc[0, 0] / (_N * _D)


def _tc_distance_argmin(z, embedding):
    return pl.pallas_call(
        _tc_body,
        grid=(_NT,),
        in_specs=[
            pl.BlockSpec((1, _D, _T), lambda i: (i, 0, 0)),
            pl.BlockSpec((_K, _D), lambda i: (0, 0)),
        ],
        out_specs=[
            pl.BlockSpec((1, 1, _T), lambda i: (i, 0, 0)),
            pl.BlockSpec(memory_space=pltpu.SMEM),
        ],
        out_shape=[
            jax.ShapeDtypeStruct((_NT, 1, _T), jnp.int32),
            jax.ShapeDtypeStruct((1, 1), jnp.float32),
        ],
        scratch_shapes=[
            pltpu.VMEM((_K, _D), jnp.float32),
            pltpu.VMEM((_K, 1), jnp.float32),
            pltpu.SMEM((1, 1), jnp.float32),
        ],
    )(z, embedding)


def _sc_gather_hist(embedding, idx):
    info = plsc.get_sparse_core_info()
    nc, ns, nl = info.num_cores, info.num_subcores, info.num_lanes
    b_per_w = _N // (nc * ns)
    mesh = plsc.VectorSubcoreMesh(core_axis_name="c", subcore_axis_name="s")

    @functools.partial(
        pl.kernel, mesh=mesh,
        out_type=[
            jax.ShapeDtypeStruct((_N, _D), jnp.float32),
            jax.ShapeDtypeStruct((nc, _K), jnp.float32),
        ],
        scratch_types=[
            pltpu.VMEM((b_per_w,), jnp.int32),
            pltpu.VMEM((b_per_w, _D), jnp.float32),
            pltpu.VMEM((b_per_w,), jnp.float32),
            pltpu.VMEM((_K // 16,), jnp.float32),
            pltpu.VMEM_SHARED((_K,), jnp.float32),
            pltpu.SemaphoreType.DMA,
        ],
    )
    def k(table_hbm, idx_hbm, out_hbm, hist_hbm,
          idx_v, rows_v, ones_v, zer_v, hist_s, sem):
        cidx = lax.axis_index("c")
        sidx = lax.axis_index("s")
        wid = sidx * nc + cidx
        base = wid * b_per_w
        tpb = _T // b_per_w
        pltpu.sync_copy(
            idx_hbm.at[wid // tpb, 0, pl.ds((wid % tpb) * b_per_w, b_per_w)],
            idx_v)
        cp = pltpu.async_copy(table_hbm.at[idx_v], rows_v, sem)

        ones = jnp.full((nl,), 1.0, jnp.float32)

        def obody(g, carry):
            ones_v[pl.ds(g * nl, nl)] = ones
            return carry

        lax.fori_loop(0, b_per_w // nl, obody, 0)

        zeros = jnp.zeros((nl,), jnp.float32)
        zslice = _K // ns

        def zbody(g, carry):
            zer_v[pl.ds(g * nl, nl)] = zeros
            return carry

        lax.fori_loop(0, zslice // nl, zbody, 0)
        pltpu.sync_copy(zer_v, hist_s.at[pl.ds(sidx * zslice, zslice)])

        plsc.subcore_barrier()
        pltpu.sync_copy(ones_v, hist_s.at[idx_v], add=True)
        plsc.subcore_barrier()

        @pl.when(sidx == 0)
        def _():
            pltpu.sync_copy(hist_s, hist_hbm.at[cidx])

        cp.wait()
        pltpu.sync_copy(rows_v, out_hbm.at[pl.ds(base, b_per_w)])

    return k(embedding, idx)


def _fin_body(h_ref, perp_ref):
    counts = jnp.sum(h_ref[...], axis=0, keepdims=True)
    avg = counts / _N
    ent = jnp.sum(avg * jnp.log(avg + 1e-10))
    perp_ref[0, 0] = jnp.exp(-ent)


def _finalize_perp(hists):
    return pl.pallas_call(
        _fin_body,
        out_specs=pl.BlockSpec(memory_space=pltpu.SMEM),
        out_shape=jax.ShapeDtypeStruct((1, 1), jnp.float32),
    )(hists)


def kernel(z, embedding):
    B, D, T = z.shape
    idx3, loss = _tc_distance_argmin(z, embedding)
    z_vq, hists = _sc_gather_hist(embedding, idx3)
    perp = _finalize_perp(hists)
    z_out = jnp.transpose(z_vq.reshape(B, T, D), (0, 2, 1))
    scalar_loss = loss[0, 0]
    return (z_out, scalar_loss, scalar_loss, perp[0, 0])
